# baseline (speedup 1.0000x reference)
import functools

import jax
import jax.numpy as jnp
from jax import lax
from jax.experimental import pallas as pl
from jax.experimental.pallas import tpu as pltpu
from jax.experimental.pallas import tpu_sc as plsc

D = 64
SCALE = 0.125
NUM_WORKERS = 32
CHUNK_ROWS = 2
ROW_UNROLL = 4


def kernel(inputs, table):
    B, S = inputs.shape
    idx = inputs.reshape(-1)
    n = B * S
    chunk = CHUNK_ROWS * S
    rows_per_worker = B // NUM_WORKERS
    per_worker = rows_per_worker * S
    steps = rows_per_worker // CHUNK_ROWS
    assert B % NUM_WORKERS == 0 and rows_per_worker % CHUNK_ROWS == 0
    assert steps % 2 == 0 and chunk % 8 == 0

    mesh = plsc.VectorSubcoreMesh(core_axis_name="c", subcore_axis_name="s")

    @functools.partial(
        pl.kernel,
        out_type=jax.ShapeDtypeStruct((B, S, D), jnp.float32),
        mesh=mesh,
        compiler_params=pltpu.CompilerParams(use_tc_tiling_on_sc=False),
        scratch_types=[
            pltpu.VMEM((per_worker,), jnp.int32),
            pltpu.VMEM((chunk, D), jnp.float32),
            pltpu.VMEM((chunk, D), jnp.float32),
            pltpu.VMEM((CHUNK_ROWS, S, D), jnp.float32),
            pltpu.VMEM((CHUNK_ROWS, S, D), jnp.float32),
            pltpu.SemaphoreType.DMA,
            pltpu.SemaphoreType.DMA,
            pltpu.SemaphoreType.DMA,
            pltpu.SemaphoreType.DMA,
        ],
    )
    def emb(table_hbm, idx_hbm, out_hbm, idx_v, g0, g1, o0, o1,
            gsem0, gsem1, osem0, osem1):
        gbuf = (g0, g1)
        obuf = (o0, o1)
        gsem = (gsem0, gsem1)
        osem = (osem0, osem1)
        wid = lax.axis_index("s") * 2 + lax.axis_index("c")
        base = wid * per_worker
        base_row = wid * rows_per_worker

        pltpu.sync_copy(idx_hbm.at[pl.ds(base, per_worker)], idx_v)

        def gather_start(cur, b):
            src = table_hbm.at[idx_v.at[pl.ds(cur * chunk, chunk)]]
            pltpu.make_async_copy(src, gbuf[b], gsem[b]).start()

        def gather_wait(cur, b):
            src = table_hbm.at[idx_v.at[pl.ds(cur * chunk, chunk)]]
            pltpu.make_async_copy(src, gbuf[b], gsem[b]).wait()

        def out_start(cur, b):
            dst = out_hbm.at[pl.ds(base_row + cur * CHUNK_ROWS, CHUNK_ROWS)]
            pltpu.make_async_copy(obuf[b], dst, osem[b]).start()

        def out_wait(cur, b):
            dst = out_hbm.at[pl.ds(base_row + cur * CHUNK_ROWS, CHUNK_ROWS)]
            pltpu.make_async_copy(obuf[b], dst, osem[b]).wait()

        gather_start(0, 0)
        gather_start(1, 1)

        @pl.loop(0, steps, step=2)
        def _(s):
            for b in range(2):
                cur = s + b
                gather_wait(cur, b)

                @pl.when(cur >= 2)
                def _():
                    out_wait(cur - 2, b)

                for r in range(CHUNK_ROWS):
                    @pl.loop(0, S, step=ROW_UNROLL)
                    def _(j, r=r):
                        for u in range(ROW_UNROLL):
                            src = gbuf[b].at[r * S + j + u]
                            dst = obuf[b].at[r, j + u]
                            for c in range(0, D, 16):
                                dst[pl.ds(c, 16)] = src[pl.ds(c, 16)] * SCALE

                out_start(cur, b)

                @pl.when(cur + 2 < steps)
                def _():
                    gather_start(cur + 2, b)

        out_wait(steps - 2, 0)
        out_wait(steps - 1, 1)

    return emb(table, idx)


# speedup vs baseline: 1.3284x; 1.3284x over previous
import functools

import jax
import jax.numpy as jnp
from jax import lax
from jax.experimental import pallas as pl
from jax.experimental.pallas import tpu as pltpu
from jax.experimental.pallas import tpu_sc as plsc

D = 64
DPAD = 128
SCALE = 0.125
NUM_WORKERS = 32
ROW_UNROLL = 4


def kernel(inputs, table):
    B, S = inputs.shape
    rows_per_worker = B // NUM_WORKERS
    assert B % NUM_WORKERS == 0 and rows_per_worker % 2 == 0 and S % 8 == 0

    mesh = plsc.VectorSubcoreMesh(core_axis_name="c", subcore_axis_name="s")

    @functools.partial(
        pl.kernel,
        out_type=jax.ShapeDtypeStruct((B, S, DPAD), jnp.float32),
        mesh=mesh,
        compiler_params=pltpu.CompilerParams(use_tc_tiling_on_sc=False),
        scratch_types=[
            pltpu.VMEM((rows_per_worker, S), jnp.int32),
            pltpu.VMEM((S, D), jnp.float32),
            pltpu.VMEM((S, D), jnp.float32),
            pltpu.VMEM((S, D), jnp.float32),
            pltpu.VMEM((S, D), jnp.float32),
            pltpu.SemaphoreType.DMA,
            pltpu.SemaphoreType.DMA,
            pltpu.SemaphoreType.DMA,
            pltpu.SemaphoreType.DMA,
        ],
    )
    def emb(table_hbm, idx_hbm, out_hbm, idx_v, g0, g1, o0, o1,
            gsem0, gsem1, osem0, osem1):
        gbuf = (g0, g1)
        obuf = (o0, o1)
        gsem = (gsem0, gsem1)
        osem = (osem0, osem1)
        wid = lax.axis_index("s") * 2 + lax.axis_index("c")
        base_row = wid * rows_per_worker

        pltpu.sync_copy(idx_hbm.at[pl.ds(base_row, rows_per_worker)], idx_v)

        def gather_start(r, b):
            src = table_hbm.at[idx_v.at[r]]
            pltpu.make_async_copy(src, gbuf[b], gsem[b]).start()

        def gather_wait(r, b):
            src = table_hbm.at[idx_v.at[r]]
            pltpu.make_async_copy(src, gbuf[b], gsem[b]).wait()

        def out_start(r, b):
            dst = out_hbm.at[base_row + r, :, pl.ds(0, D)]
            pltpu.make_async_copy(obuf[b], dst, osem[b]).start()

        def out_wait(r, b):
            dst = out_hbm.at[base_row + r, :, pl.ds(0, D)]
            pltpu.make_async_copy(obuf[b], dst, osem[b]).wait()

        gather_start(0, 0)
        gather_start(1, 1)

        @pl.loop(0, rows_per_worker, step=2)
        def _(s):
            for b in range(2):
                cur = s + b
                gather_wait(cur, b)

                @pl.when(cur >= 2)
                def _():
                    out_wait(cur - 2, b)

                @pl.loop(0, S, step=ROW_UNROLL)
                def _(j):
                    for u in range(ROW_UNROLL):
                        src = gbuf[b].at[j + u]
                        dst = obuf[b].at[j + u]
                        for c in range(0, D, 16):
                            dst[pl.ds(c, 16)] = src[pl.ds(c, 16)] * SCALE

                out_start(cur, b)

                @pl.when(cur + 2 < rows_per_worker)
                def _():
                    gather_start(cur + 2, b)

        out_wait(rows_per_worker - 2, 0)
        out_wait(rows_per_worker - 1, 1)

    out = emb(table, inputs)
    return out[:, :, :D]
